# baseline (speedup 1.0000x reference)
import jax
import jax.numpy as jnp
from jax import lax
from jax.experimental import pallas as pl
from jax.experimental.pallas import tpu as pltpu
from jax.experimental.pallas import tpu_sc as plsc

_NC, _NS = 2, 16
_NW = _NC * _NS


def _proj_body(obj_ref, ws_ref, bs_ref, wo_ref, bo_ref, s_ref, o_ref):
    x = obj_ref[...]
    s_ref[...] = jnp.dot(x, ws_ref[...],
                         preferred_element_type=jnp.float32) + bs_ref[...]
    o_ref[...] = jnp.dot(x, wo_ref[...],
                         preferred_element_type=jnp.float32) + bo_ref[...]


def _project(obj, Ws, bs, Wo, bo):
    n, d = obj.shape
    out = jax.ShapeDtypeStruct((n, d), jnp.float32)
    return pl.pallas_call(_proj_body, out_shape=(out, out))(
        obj, Ws, bs.reshape(1, d), Wo, bo.reshape(1, d))


def _edge_product(s, o, u, iarr, jarr):
    n, d = s.shape
    e = u.shape[0]
    epw = e // _NW
    ch = 16
    chn = epw // ch
    nrt = n // _NS

    def body(s_hbm, o_hbm, u_hbm, i_hbm, j_hbm, t_hbm,
             ivall, jvall, ivc, jvc, sv, ov, uv, tv,
             stab, otab, gsem, usem):
        c = lax.axis_index("c")
        s_id = lax.axis_index("s")
        wid = s_id * _NC + c
        ebase = wid * epw
        pltpu.sync_copy(s_hbm.at[pl.ds(s_id * nrt, nrt)],
                        stab.at[pl.ds(s_id * nrt, nrt)])
        pltpu.sync_copy(o_hbm.at[pl.ds(s_id * nrt, nrt)],
                        otab.at[pl.ds(s_id * nrt, nrt)])
        pltpu.sync_copy(i_hbm.at[pl.ds(ebase, epw)], ivall)
        pltpu.sync_copy(j_hbm.at[pl.ds(ebase, epw)], jvall)
        plsc.subcore_barrier()

        def issue(it, slot):
            base = it * ch
            for q in range(ch // 16):
                ivc[slot, pl.ds(q * 16, 16)] = ivall[pl.ds(base + q * 16, 16)]
                jvc[slot, pl.ds(q * 16, 16)] = jvall[pl.ds(base + q * 16, 16)]
            pltpu.async_copy(stab.at[ivc.at[slot]], sv.at[slot], gsem)
            pltpu.async_copy(otab.at[jvc.at[slot]], ov.at[slot], gsem)
            pltpu.async_copy(u_hbm.at[pl.ds(ebase + base, ch)],
                             uv.at[slot], usem)

        def drain(slot):
            pltpu.make_async_copy(stab.at[ivc.at[slot]],
                                  sv.at[slot], gsem).wait()
            pltpu.make_async_copy(otab.at[jvc.at[slot]],
                                  ov.at[slot], gsem).wait()
            pltpu.make_async_copy(u_hbm.at[pl.ds(0, ch)],
                                  uv.at[slot], usem).wait()

        issue(0, 0)

        def pair(g, carry):
            for b in range(2):
                it = g * 2 + b

                @pl.when(it + 1 < chn)
                def _():
                    issue(it + 1, 1 - b)

                drain(b)

                def row(r, c2):
                    for q in range(d // 16):
                        sl = pl.ds(q * 16, 16)
                        tv[r, sl] = sv[b, r, sl] * ov[b, r, sl] * uv[b, r, sl]
                    return c2

                lax.fori_loop(0, ch, row, 0)
                pltpu.sync_copy(tv, t_hbm.at[pl.ds(ebase + it * ch, ch)])
            return carry

        lax.fori_loop(0, chn // 2, pair, 0)

    mesh = plsc.VectorSubcoreMesh(core_axis_name="c", subcore_axis_name="s")
    fn = pl.kernel(
        body,
        out_type=jax.ShapeDtypeStruct((e, d), jnp.float32),
        mesh=mesh,
        compiler_params=pltpu.CompilerParams(use_tc_tiling_on_sc=False),
        scratch_types=[
            pltpu.VMEM((epw,), jnp.int32),
            pltpu.VMEM((epw,), jnp.int32),
            pltpu.VMEM((2, ch), jnp.int32),
            pltpu.VMEM((2, ch), jnp.int32),
            pltpu.VMEM((2, ch, d), jnp.float32),
            pltpu.VMEM((2, ch, d), jnp.float32),
            pltpu.VMEM((2, ch, d), jnp.float32),
            pltpu.VMEM((ch, d), jnp.float32),
            pltpu.VMEM_SHARED((n, d), jnp.float32),
            pltpu.VMEM_SHARED((n, d), jnp.float32),
            pltpu.SemaphoreType.DMA,
            pltpu.SemaphoreType.DMA,
        ],
    )
    return fn(s, o, u, iarr, jarr)


def _attf_body(t_ref, ww_ref, bw_ref, out_ref):
    out_ref[...] = jnp.dot(t_ref[...], ww_ref[...],
                           preferred_element_type=jnp.float32) + bw_ref[...]


def _atten_feats(t, Ww, bw):
    e, d = t.shape
    p = Ww.shape[1]
    blk = 2048
    return pl.pallas_call(
        _attf_body,
        grid=(e // blk,),
        in_specs=[
            pl.BlockSpec((blk, d), lambda i: (i, 0)),
            pl.BlockSpec((d, p), lambda i: (0, 0)),
            pl.BlockSpec((1, p), lambda i: (0, 0)),
        ],
        out_specs=pl.BlockSpec((blk, p), lambda i: (i, 0)),
        out_shape=jax.ShapeDtypeStruct((e, p), jnp.float32),
    )(t, Ww, bw.reshape(1, p))


def _scatter_softmax(attf, iarr, jarr, n):
    e, p = attf.shape
    rows = 32
    rsh = rows.bit_length() - 1
    nblk_per_sc = n // rows // _NC
    sprows = rows * n
    ept = e // _NS
    zrows = 1024
    rpt = rows // _NS

    def body(attf_hbm, i_hbm, j_hbm, z_hbm, av, iv, jv, xv, zerov, rbuf, spbuf):
        c = lax.axis_index("c")
        s = lax.axis_index("s")
        tb = s * ept
        pltpu.sync_copy(attf_hbm.at[pl.ds(tb, ept)], av)
        pltpu.sync_copy(i_hbm.at[pl.ds(tb, ept)], iv)
        pltpu.sync_copy(j_hbm.at[pl.ds(tb, ept)], jv)

        def zrow(r, carry):
            zerov[r, :] = jnp.zeros((16,), jnp.float32)
            return carry

        lax.fori_loop(0, zrows, zrow, 0)

        for bb in range(nblk_per_sc):
            b = c * nblk_per_sc + bb
            for q in range(sprows // _NS // zrows):
                pltpu.sync_copy(
                    zerov, spbuf.at[pl.ds(s * (sprows // _NS) + q * zrows,
                                          zrows)])
            plsc.subcore_barrier()

            def ixc(k, carry):
                for u2 in range(4):
                    sl = pl.ds((k * 4 + u2) * 16, 16)
                    iv16 = iv[sl]
                    jv16 = jv[sl]
                    valid = (iv16 >> rsh) == b
                    loc = ((iv16 & (rows - 1)) << 10) | jv16
                    xv[sl] = jnp.where(valid, loc, sprows)
                return carry

            lax.fori_loop(0, ept // 64, ixc, 0)
            pltpu.sync_copy(av, spbuf.at[xv], add=True)
            plsc.subcore_barrier()

            for q in range(rpt):
                i_loc = s * rpt + q
                i_g = b * rows + i_loc
                pltpu.sync_copy(spbuf.at[pl.ds(i_loc * n, n)], rbuf)
                rbuf[i_g, :] = rbuf[i_g, :] - 1e4

                def mx(j, m):
                    m2 = m
                    for u2 in range(8):
                        m2 = jnp.maximum(m2, rbuf[j * 8 + u2, :])
                    return m2

                m = lax.fori_loop(0, n // 8, mx, rbuf[0, :])

                def ex(j, acc):
                    a2 = acc
                    for u2 in range(4):
                        v = jnp.exp(rbuf[j * 4 + u2, :] - m)
                        rbuf[j * 4 + u2, :] = v
                        a2 = a2 + v
                    return a2

                acc = lax.fori_loop(0, n // 4, ex, jnp.zeros((16,), jnp.float32))
                r = 1.0 / acc

                def sc(j, carry):
                    for u2 in range(8):
                        rbuf[j * 8 + u2, :] = rbuf[j * 8 + u2, :] * r
                    return carry

                lax.fori_loop(0, n // 8, sc, 0)
                pltpu.sync_copy(rbuf, z_hbm.at[pl.ds(i_g * n, n)])
            plsc.subcore_barrier()

    mesh = plsc.VectorSubcoreMesh(core_axis_name="c", subcore_axis_name="s")
    fn = pl.kernel(
        body,
        out_type=jax.ShapeDtypeStruct((n * n, p), jnp.float32),
        mesh=mesh,
        compiler_params=pltpu.CompilerParams(use_tc_tiling_on_sc=False),
        scratch_types=[
            pltpu.VMEM((ept, p), jnp.float32),
            pltpu.VMEM((ept,), jnp.int32),
            pltpu.VMEM((ept,), jnp.int32),
            pltpu.VMEM((ept,), jnp.int32),
            pltpu.VMEM((zrows, p), jnp.float32),
            pltpu.VMEM((n, p), jnp.float32),
            pltpu.VMEM_SHARED((sprows + 8, p), jnp.float32),
        ],
    )
    return fn(attf, iarr, jarr)


def _softmax_rows(z2, n, p):
    npcols = z2.shape[1]
    bi = 16
    nch = npcols // 128

    def body(z_ref, o_ref):
        pid = pl.program_id(0)
        rowv = pid * bi + lax.broadcasted_iota(jnp.int32, (bi, 128), 0)
        colb = lax.broadcasted_iota(jnp.int32, (bi, 128), 1)

        def masked(k):
            x = z_ref[:, k * 128:(k + 1) * 128]
            j = (k * 128 + colb) >> 4
            return x - jnp.where(j == rowv, 1e4, 0.0).astype(jnp.float32)

        m = masked(0)
        for k in range(1, nch):
            m = jnp.maximum(m, masked(k))
        for sft in (16, 32, 64):
            m = jnp.maximum(
                m, jnp.concatenate([m[:, sft:], m[:, :sft]], axis=1))
        ssum = jnp.zeros((bi, 128), jnp.float32)
        for k in range(nch):
            ex = jnp.exp(masked(k) - m)
            ssum = ssum + ex
            o_ref[:, k * 128:(k + 1) * 128] = ex
        for sft in (16, 32, 64):
            ssum = ssum + jnp.concatenate(
                [ssum[:, sft:], ssum[:, :sft]], axis=1)
        r = 1.0 / ssum
        for k in range(nch):
            o_ref[:, k * 128:(k + 1) * 128] = (
                o_ref[:, k * 128:(k + 1) * 128] * r)

    return pl.pallas_call(
        body,
        grid=(n // bi,),
        in_specs=[pl.BlockSpec((bi, npcols), lambda i: (i, 0))],
        out_specs=pl.BlockSpec((bi, npcols), lambda i: (i, 0)),
        out_shape=jax.ShapeDtypeStruct((n, npcols), jnp.float32),
    )(z2)


def kernel(obj_feats, union_feats, pair_idxs, Ws, bs, Wo, bo, Ww, bw):
    n, d = obj_feats.shape
    p = Ww.shape[1]
    s, o = _project(obj_feats, Ws, bs, Wo, bo)
    pair_t = pair_idxs.T
    iarr = pair_t[0]
    jarr = pair_t[1]
    t = _edge_product(s, o, union_feats, iarr, jarr)
    attf = _atten_feats(t, Ww, bw)
    z = _scatter_softmax(attf, iarr, jarr, n)
    return z.reshape(n, n, p)


# speedup vs baseline: 1.5681x; 1.5681x over previous
import jax
import jax.numpy as jnp
from jax import lax
from jax.experimental import pallas as pl
from jax.experimental.pallas import tpu as pltpu
from jax.experimental.pallas import tpu_sc as plsc

_NC, _NS = 2, 16
_NW = _NC * _NS


def _proj_body(obj_ref, ws_ref, bs_ref, wo_ref, bo_ref, s_ref, o_ref):
    x = obj_ref[...]
    s_ref[...] = jnp.dot(x, ws_ref[...], precision=jax.lax.Precision.HIGHEST,
                         preferred_element_type=jnp.float32) + bs_ref[...]
    o_ref[...] = jnp.dot(x, wo_ref[...], precision=jax.lax.Precision.HIGHEST,
                         preferred_element_type=jnp.float32) + bo_ref[...]


def _project(obj, Ws, bs, Wo, bo):
    n, d = obj.shape
    out = jax.ShapeDtypeStruct((n, d), jnp.float32)
    return pl.pallas_call(_proj_body, out_shape=(out, out))(
        obj, Ws, bs.reshape(1, d), Wo, bo.reshape(1, d))


def _edge_product(s, o, u, pair):
    n, d = s.shape
    e = u.shape[0]
    epw = e // _NW
    ch = 16
    chn = epw // ch
    nrt = n // _NS

    def body(s_hbm, o_hbm, u_hbm, pair_hbm, t_hbm,
             pvall, ivc, jvc, sv, ov, uv,
             stab, otab, gsem, usem):
        c = lax.axis_index("c")
        s_id = lax.axis_index("s")
        wid = s_id * _NC + c
        ebase = wid * epw
        pltpu.sync_copy(s_hbm.at[pl.ds(s_id * nrt, nrt)],
                        stab.at[pl.ds(s_id * nrt, nrt)])
        pltpu.sync_copy(o_hbm.at[pl.ds(s_id * nrt, nrt)],
                        otab.at[pl.ds(s_id * nrt, nrt)])
        pltpu.sync_copy(pair_hbm.at[pl.ds(ebase, epw)], pvall)
        plsc.subcore_barrier()

        lanes = lax.iota(jnp.int32, 16)

        def issue(it, slot):
            base = it * ch
            for q in range(ch // 16):
                rows_ix = base + q * 16 + lanes
                ivc[slot, pl.ds(q * 16, 16)] = plsc.load_gather(
                    pvall, [rows_ix, jnp.zeros((16,), jnp.int32)])
                jvc[slot, pl.ds(q * 16, 16)] = plsc.load_gather(
                    pvall, [rows_ix, jnp.ones((16,), jnp.int32)])
            pltpu.async_copy(stab.at[ivc.at[slot]], sv.at[slot], gsem)
            pltpu.async_copy(otab.at[jvc.at[slot]], ov.at[slot], gsem)
            pltpu.async_copy(u_hbm.at[pl.ds(ebase + base, ch)],
                             uv.at[slot], usem)

        def drain(slot):
            pltpu.make_async_copy(stab.at[ivc.at[slot]],
                                  sv.at[slot], gsem).wait()
            pltpu.make_async_copy(otab.at[jvc.at[slot]],
                                  ov.at[slot], gsem).wait()
            pltpu.make_async_copy(u_hbm.at[pl.ds(0, ch)],
                                  uv.at[slot], usem).wait()

        issue(0, 0)

        def pair(g, carry):
            for b in range(2):
                it = g * 2 + b

                @pl.when(it + 1 < chn)
                def _():
                    issue(it + 1, 1 - b)

                drain(b)

                def row(r, c2):
                    for q in range(d // 16):
                        sl = pl.ds(q * 16, 16)
                        sv[b, r, sl] = sv[b, r, sl] * ov[b, r, sl] * uv[b, r, sl]
                    return c2

                lax.fori_loop(0, ch, row, 0)
                pltpu.sync_copy(sv.at[b], t_hbm.at[pl.ds(ebase + it * ch, ch)])
            return carry

        lax.fori_loop(0, chn // 2, pair, 0)

    mesh = plsc.VectorSubcoreMesh(core_axis_name="c", subcore_axis_name="s")
    fn = pl.kernel(
        body,
        out_type=jax.ShapeDtypeStruct((e, d), jnp.float32),
        mesh=mesh,
        compiler_params=pltpu.CompilerParams(use_tc_tiling_on_sc=False, needs_layout_passes=False),
        scratch_types=[
            pltpu.VMEM((epw, 2), jnp.int32),
            pltpu.VMEM((2, ch), jnp.int32),
            pltpu.VMEM((2, ch), jnp.int32),
            pltpu.VMEM((2, ch, d), jnp.float32),
            pltpu.VMEM((2, ch, d), jnp.float32),
            pltpu.VMEM((2, ch, d), jnp.float32),
            pltpu.VMEM_SHARED((n, d), jnp.float32),
            pltpu.VMEM_SHARED((n, d), jnp.float32),
            pltpu.SemaphoreType.DMA,
            pltpu.SemaphoreType.DMA,
        ],
    )
    return fn(s, o, u, pair)


def _attf_body(t_ref, ww_ref, bw_ref, out_ref):
    out_ref[...] = jnp.dot(t_ref[...], ww_ref[...],
                           precision=jax.lax.Precision.HIGHEST,
                           preferred_element_type=jnp.float32) + bw_ref[...]


def _atten_feats(t, Ww, bw):
    e, d = t.shape
    p = Ww.shape[1]
    blk = 2048
    return pl.pallas_call(
        _attf_body,
        grid=(e // blk,),
        in_specs=[
            pl.BlockSpec((blk, d), lambda i: (i, 0)),
            pl.BlockSpec((d, p), lambda i: (0, 0)),
            pl.BlockSpec((1, p), lambda i: (0, 0)),
        ],
        out_specs=pl.BlockSpec((blk, p), lambda i: (i, 0)),
        out_shape=jax.ShapeDtypeStruct((e, p), jnp.float32),
    )(t, Ww, bw.reshape(1, p))


def _scatter_dense(attf, pair, n, half, halves):
    e, p = attf.shape
    nh = n // halves
    rows = 64
    rsh = rows.bit_length() - 1
    nblk_per_sc = nh // rows // _NC
    sprows = rows * n
    ept = e // _NS
    zrows = 512

    def body(attf_hbm, pair_hbm, z_hbm, av, pv, xv, zerov, dix, dvals, spbuf):
        c = lax.axis_index("c")
        s = lax.axis_index("s")
        tb = s * ept
        pltpu.sync_copy(attf_hbm.at[pl.ds(tb, ept)], av)
        pltpu.sync_copy(pair_hbm.at[pl.ds(tb, ept)], pv)
        lanes = lax.iota(jnp.int32, 16)
        zl = jnp.zeros((16,), jnp.int32)
        ol = jnp.ones((16,), jnp.int32)

        def zrow(r, carry):
            zerov[r, :] = jnp.zeros((16,), jnp.float32)
            return carry

        lax.fori_loop(0, zrows, zrow, 0)

        def drow(r, carry):
            dvals[r, :] = jnp.full((16,), -1e4, jnp.float32)
            return carry

        lax.fori_loop(0, 16, drow, 0)

        for bb in range(nblk_per_sc):
            bl = c * nblk_per_sc + bb
            b = half * (nh // rows) + bl
            for q in range(sprows // _NS // zrows):
                pltpu.sync_copy(
                    zerov, spbuf.at[pl.ds(s * (sprows // _NS) + q * zrows,
                                          zrows)])
            plsc.subcore_barrier()

            def ixc(k, carry):
                for u2 in range(4):
                    sl = pl.ds((k * 4 + u2) * 16, 16)
                    rows_ix = (k * 4 + u2) * 16 + lanes
                    iv16 = plsc.load_gather(pv, [rows_ix, zl])
                    jv16 = plsc.load_gather(pv, [rows_ix, ol])
                    valid = (iv16 >> rsh) == b
                    loc = ((iv16 & (rows - 1)) << 10) | jv16
                    xv[sl] = jnp.where(valid, loc, sprows)
                return carry

            lax.fori_loop(0, ept // 64, ixc, 0)
            pltpu.sync_copy(av, spbuf.at[xv], add=True)
            i_loc = s * (rows // _NS) + lanes
            dloc = (i_loc << 10) + b * rows + i_loc
            dix[:] = jnp.where(lanes < rows // _NS, dloc, sprows)
            pltpu.sync_copy(dvals, spbuf.at[dix], add=True)
            plsc.subcore_barrier()
            pltpu.sync_copy(
                spbuf.at[pl.ds(s * (sprows // _NS), sprows // _NS)],
                z_hbm.at[pl.ds(bl * sprows + s * (sprows // _NS),
                               sprows // _NS)])
            plsc.subcore_barrier()

    mesh = plsc.VectorSubcoreMesh(core_axis_name="c", subcore_axis_name="s")
    fn = pl.kernel(
        body,
        out_type=jax.ShapeDtypeStruct((nh * n, p), jnp.float32),
        mesh=mesh,
        compiler_params=pltpu.CompilerParams(use_tc_tiling_on_sc=False,
                                             needs_layout_passes=False),
        scratch_types=[
            pltpu.VMEM((ept, p), jnp.float32),
            pltpu.VMEM((ept, 2), jnp.int32),
            pltpu.VMEM((ept,), jnp.int32),
            pltpu.VMEM((zrows, p), jnp.float32),
            pltpu.VMEM((16,), jnp.int32),
            pltpu.VMEM((16, p), jnp.float32),
            pltpu.VMEM_SHARED((sprows + 8, p), jnp.float32),
        ],
    )
    return fn(attf, pair)


def _softmax_rows(z2, n, p):
    npcols = z2.shape[1]
    bi = 16
    nch = npcols // 128

    def body(z_ref, o_ref):
        m = z_ref[:, 0:128]
        for k in range(1, nch):
            m = jnp.maximum(m, z_ref[:, k * 128:(k + 1) * 128])
        for sft in (16, 32, 64):
            m = jnp.maximum(
                m, jnp.concatenate([m[:, sft:], m[:, :sft]], axis=1))
        ssum = jnp.zeros((bi, 128), jnp.float32)
        for k in range(nch):
            ex = jnp.exp(z_ref[:, k * 128:(k + 1) * 128] - m)
            ssum = ssum + ex
            o_ref[:, k * 128:(k + 1) * 128] = ex
        for sft in (16, 32, 64):
            ssum = ssum + jnp.concatenate(
                [ssum[:, sft:], ssum[:, :sft]], axis=1)
        r = 1.0 / ssum
        for k in range(nch):
            o_ref[:, k * 128:(k + 1) * 128] = (
                o_ref[:, k * 128:(k + 1) * 128] * r)

    nr = z2.shape[0]
    return pl.pallas_call(
        body,
        grid=(nr // bi,),
        in_specs=[pl.BlockSpec((bi, npcols), lambda i: (i, 0))],
        out_specs=pl.BlockSpec((bi, npcols), lambda i: (i, 0)),
        out_shape=jax.ShapeDtypeStruct((nr, npcols), jnp.float32),
    )(z2)


def kernel(obj_feats, union_feats, pair_idxs, Ws, bs, Wo, bo, Ww, bw):
    n, d = obj_feats.shape
    p = Ww.shape[1]
    s, o = _project(obj_feats, Ws, bs, Wo, bo)
    t = _edge_product(s, o, union_feats, pair_idxs)
    attf = _atten_feats(t, Ww, bw)
    z0 = _scatter_dense(attf, pair_idxs, n, 0, 2)
    z1 = _scatter_dense(attf, pair_idxs, n, 1, 2)
    o0 = _softmax_rows(z0.reshape(n // 2, n * p), n, p)
    o1 = _softmax_rows(z1.reshape(n // 2, n * p), n, p)
    return jnp.concatenate([o0, o1], axis=0).reshape(n, n, p)


# speedup vs baseline: 2.1024x; 1.3407x over previous
import jax
import jax.numpy as jnp
from jax import lax
from jax.experimental import pallas as pl
from jax.experimental.pallas import tpu as pltpu
from jax.experimental.pallas import tpu_sc as plsc

_NC, _NS = 2, 16
_NW = _NC * _NS


def _proj_body(obj_ref, ws_ref, bs_ref, wo_ref, bo_ref, s_ref, o_ref):
    x = obj_ref[...]
    s_ref[...] = jnp.dot(x, ws_ref[...], precision=jax.lax.Precision.HIGHEST,
                         preferred_element_type=jnp.float32) + bs_ref[...]
    o_ref[...] = jnp.dot(x, wo_ref[...], precision=jax.lax.Precision.HIGHEST,
                         preferred_element_type=jnp.float32) + bo_ref[...]


def _project(obj, Ws, bs, Wo, bo):
    n, d = obj.shape
    out = jax.ShapeDtypeStruct((n, d), jnp.float32)
    return pl.pallas_call(_proj_body, out_shape=(out, out))(
        obj, Ws, bs.reshape(1, d), Wo, bo.reshape(1, d))


def _edge_product(s, o, u, pair):
    n, d = s.shape
    e = u.shape[0]
    epw = e // _NW
    ch = 16
    chn = epw // ch
    nrt = n // _NS

    def body(s_hbm, o_hbm, u_hbm, pair_hbm, t_hbm,
             pvall, ivc, jvc, sv, ov, uv,
             stab, otab, gsem, usem):
        c = lax.axis_index("c")
        s_id = lax.axis_index("s")
        wid = s_id * _NC + c
        ebase = wid * epw
        pltpu.sync_copy(s_hbm.at[pl.ds(s_id * nrt, nrt)],
                        stab.at[pl.ds(s_id * nrt, nrt)])
        pltpu.sync_copy(o_hbm.at[pl.ds(s_id * nrt, nrt)],
                        otab.at[pl.ds(s_id * nrt, nrt)])
        pltpu.sync_copy(pair_hbm.at[pl.ds(ebase, epw)], pvall)
        plsc.subcore_barrier()

        lanes = lax.iota(jnp.int32, 16)

        def issue(it, slot):
            base = it * ch
            for q in range(ch // 16):
                rows_ix = base + q * 16 + lanes
                ivc[slot, pl.ds(q * 16, 16)] = plsc.load_gather(
                    pvall, [rows_ix, jnp.zeros((16,), jnp.int32)])
                jvc[slot, pl.ds(q * 16, 16)] = plsc.load_gather(
                    pvall, [rows_ix, jnp.ones((16,), jnp.int32)])
            pltpu.async_copy(stab.at[ivc.at[slot]], sv.at[slot], gsem)
            pltpu.async_copy(otab.at[jvc.at[slot]], ov.at[slot], gsem)
            pltpu.async_copy(u_hbm.at[pl.ds(ebase + base, ch)],
                             uv.at[slot], usem)

        def drain(slot):
            pltpu.make_async_copy(stab.at[ivc.at[slot]],
                                  sv.at[slot], gsem).wait()
            pltpu.make_async_copy(otab.at[jvc.at[slot]],
                                  ov.at[slot], gsem).wait()
            pltpu.make_async_copy(u_hbm.at[pl.ds(0, ch)],
                                  uv.at[slot], usem).wait()

        issue(0, 0)

        def pair(g, carry):
            for b in range(2):
                it = g * 2 + b

                @pl.when(it + 1 < chn)
                def _():
                    issue(it + 1, 1 - b)

                drain(b)

                def row(r, c2):
                    for q in range(d // 16):
                        sl = pl.ds(q * 16, 16)
                        sv[b, r, sl] = sv[b, r, sl] * ov[b, r, sl] * uv[b, r, sl]
                    return c2

                lax.fori_loop(0, ch, row, 0)
                pltpu.sync_copy(sv.at[b], t_hbm.at[pl.ds(ebase + it * ch, ch)])
            return carry

        lax.fori_loop(0, chn // 2, pair, 0)

    mesh = plsc.VectorSubcoreMesh(core_axis_name="c", subcore_axis_name="s")
    fn = pl.kernel(
        body,
        out_type=jax.ShapeDtypeStruct((e, d), jnp.float32),
        mesh=mesh,
        compiler_params=pltpu.CompilerParams(use_tc_tiling_on_sc=False, needs_layout_passes=False),
        scratch_types=[
            pltpu.VMEM((epw, 2), jnp.int32),
            pltpu.VMEM((2, ch), jnp.int32),
            pltpu.VMEM((2, ch), jnp.int32),
            pltpu.VMEM((2, ch, d), jnp.float32),
            pltpu.VMEM((2, ch, d), jnp.float32),
            pltpu.VMEM((2, ch, d), jnp.float32),
            pltpu.VMEM_SHARED((n, d), jnp.float32),
            pltpu.VMEM_SHARED((n, d), jnp.float32),
            pltpu.SemaphoreType.DMA,
            pltpu.SemaphoreType.DMA,
        ],
    )
    return fn(s, o, u, pair)


def _attf_body(t_ref, ww_ref, bw_ref, out_ref):
    out_ref[...] = jnp.dot(t_ref[...], ww_ref[...],
                           precision=jax.lax.Precision.HIGHEST,
                           preferred_element_type=jnp.float32) + bw_ref[...]


def _atten_feats(t, Ww, bw):
    e, d = t.shape
    p = Ww.shape[1]
    blk = 2048
    return pl.pallas_call(
        _attf_body,
        grid=(e // blk,),
        in_specs=[
            pl.BlockSpec((blk, d), lambda i: (i, 0)),
            pl.BlockSpec((d, p), lambda i: (0, 0)),
            pl.BlockSpec((1, p), lambda i: (0, 0)),
        ],
        out_specs=pl.BlockSpec((blk, p), lambda i: (i, 0)),
        out_shape=jax.ShapeDtypeStruct((e, p), jnp.float32),
    )(t, Ww, bw.reshape(1, p))


def _scatter_dense(attf, pair, n, half, halves):
    e, p = attf.shape
    nh = n // halves
    rows = 64
    rsh = rows.bit_length() - 1
    nblk_per_sc = nh // rows // _NC
    sprows = rows * n
    ept = e // _NS
    zrows = 512

    def body(attf_hbm, pair_hbm, z_hbm, av, pv, xv, zerov, dix, dvals, spbuf):
        c = lax.axis_index("c")
        s = lax.axis_index("s")
        tb = s * ept
        pltpu.sync_copy(attf_hbm.at[pl.ds(tb, ept)], av)
        pltpu.sync_copy(pair_hbm.at[pl.ds(tb, ept)], pv)
        lanes = lax.iota(jnp.int32, 16)
        zl = jnp.zeros((16,), jnp.int32)
        ol = jnp.ones((16,), jnp.int32)

        def zrow(r, carry):
            zerov[r, :] = jnp.zeros((16,), jnp.float32)
            return carry

        lax.fori_loop(0, zrows, zrow, 0)

        def drow(r, carry):
            dvals[r, :] = jnp.full((16,), -1e4, jnp.float32)
            return carry

        lax.fori_loop(0, 16, drow, 0)

        for bb in range(nblk_per_sc):
            bl = c * nblk_per_sc + bb
            b = half * (nh // rows) + bl
            for q in range(sprows // _NS // zrows):
                pltpu.sync_copy(
                    zerov, spbuf.at[pl.ds(s * (sprows // _NS) + q * zrows,
                                          zrows)])
            plsc.subcore_barrier()

            dbase = sprows + (s << 8)

            def ixc(k, carry):
                for u2 in range(4):
                    sl = pl.ds((k * 4 + u2) * 16, 16)
                    rows_ix = (k * 4 + u2) * 16 + lanes
                    iv16 = plsc.load_gather(pv, [rows_ix, zl])
                    jv16 = plsc.load_gather(pv, [rows_ix, ol])
                    valid = (iv16 >> rsh) == b
                    loc = ((iv16 & (rows - 1)) << 10) | jv16
                    dummy = dbase + (((k * 4 + u2) * 16) & 255) + lanes
                    xv[sl] = jnp.where(valid, loc, dummy)
                return carry

            lax.fori_loop(0, ept // 64, ixc, 0)
            pltpu.sync_copy(av, spbuf.at[xv], add=True)
            i_loc = s * (rows // _NS) + lanes
            dloc = (i_loc << 10) + b * rows + i_loc
            dix[:] = jnp.where(lanes < rows // _NS, dloc, dbase + lanes)
            pltpu.sync_copy(dvals, spbuf.at[dix], add=True)
            plsc.subcore_barrier()
            pltpu.sync_copy(
                spbuf.at[pl.ds(s * (sprows // _NS), sprows // _NS)],
                z_hbm.at[pl.ds(bl * sprows + s * (sprows // _NS),
                               sprows // _NS)])

    mesh = plsc.VectorSubcoreMesh(core_axis_name="c", subcore_axis_name="s")
    fn = pl.kernel(
        body,
        out_type=jax.ShapeDtypeStruct((nh * n, p), jnp.float32),
        mesh=mesh,
        compiler_params=pltpu.CompilerParams(use_tc_tiling_on_sc=False,
                                             needs_layout_passes=False),
        scratch_types=[
            pltpu.VMEM((ept, p), jnp.float32),
            pltpu.VMEM((ept, 2), jnp.int32),
            pltpu.VMEM((ept,), jnp.int32),
            pltpu.VMEM((zrows, p), jnp.float32),
            pltpu.VMEM((16,), jnp.int32),
            pltpu.VMEM((16, p), jnp.float32),
            pltpu.VMEM_SHARED((sprows + _NS * 256, p), jnp.float32),
        ],
    )
    return fn(attf, pair)


def _softmax_rows(z2, n, p):
    npcols = z2.shape[1]
    bi = 16
    nch = npcols // 128

    def body(z_ref, o_ref):
        m = z_ref[:, 0:128]
        for k in range(1, nch):
            m = jnp.maximum(m, z_ref[:, k * 128:(k + 1) * 128])
        for sft in (16, 32, 64):
            m = jnp.maximum(
                m, jnp.concatenate([m[:, sft:], m[:, :sft]], axis=1))
        ssum = jnp.zeros((bi, 128), jnp.float32)
        for k in range(nch):
            ex = jnp.exp(z_ref[:, k * 128:(k + 1) * 128] - m)
            ssum = ssum + ex
            o_ref[:, k * 128:(k + 1) * 128] = ex
        for sft in (16, 32, 64):
            ssum = ssum + jnp.concatenate(
                [ssum[:, sft:], ssum[:, :sft]], axis=1)
        r = 1.0 / ssum
        for k in range(nch):
            o_ref[:, k * 128:(k + 1) * 128] = (
                o_ref[:, k * 128:(k + 1) * 128] * r)

    nr = z2.shape[0]
    return pl.pallas_call(
        body,
        grid=(nr // bi,),
        in_specs=[pl.BlockSpec((bi, npcols), lambda i: (i, 0))],
        out_specs=pl.BlockSpec((bi, npcols), lambda i: (i, 0)),
        out_shape=jax.ShapeDtypeStruct((nr, npcols), jnp.float32),
    )(z2)


def kernel(obj_feats, union_feats, pair_idxs, Ws, bs, Wo, bo, Ww, bw):
    n, d = obj_feats.shape
    p = Ww.shape[1]
    s, o = _project(obj_feats, Ws, bs, Wo, bo)
    t = _edge_product(s, o, union_feats, pair_idxs)
    attf = _atten_feats(t, Ww, bw)
    z0 = _scatter_dense(attf, pair_idxs, n, 0, 2)
    z1 = _scatter_dense(attf, pair_idxs, n, 1, 2)
    o0 = _softmax_rows(z0.reshape(n // 2, n * p), n, p)
    o1 = _softmax_rows(z1.reshape(n // 2, n * p), n, p)
    return jnp.concatenate([o0, o1], axis=0).reshape(n, n, p)


# speedup vs baseline: 2.1026x; 1.0001x over previous
import jax
import jax.numpy as jnp
from jax import lax
from jax.experimental import pallas as pl
from jax.experimental.pallas import tpu as pltpu
from jax.experimental.pallas import tpu_sc as plsc

_NC, _NS = 2, 16
_NW = _NC * _NS


def _proj_body(obj_ref, ws_ref, bs_ref, wo_ref, bo_ref, s_ref, o_ref):
    x = obj_ref[...]
    s_ref[...] = jnp.dot(x, ws_ref[...], precision=jax.lax.Precision.HIGHEST,
                         preferred_element_type=jnp.float32) + bs_ref[...]
    o_ref[...] = jnp.dot(x, wo_ref[...], precision=jax.lax.Precision.HIGHEST,
                         preferred_element_type=jnp.float32) + bo_ref[...]


def _project(obj, Ws, bs, Wo, bo):
    n, d = obj.shape
    out = jax.ShapeDtypeStruct((n, d), jnp.float32)
    return pl.pallas_call(_proj_body, out_shape=(out, out))(
        obj, Ws, bs.reshape(1, d), Wo, bo.reshape(1, d))


def _edge_product(s, o, u, pair):
    n, d = s.shape
    e = u.shape[0]
    epw = e // _NW
    ch = 16
    chn = epw // ch
    nrt = n // _NS

    def body(s_hbm, o_hbm, u_hbm, pair_hbm, t_hbm,
             pvall, ivc, jvc, sv, ov, uv,
             stab, otab, gsem, usem):
        c = lax.axis_index("c")
        s_id = lax.axis_index("s")
        wid = s_id * _NC + c
        ebase = wid * epw
        pltpu.sync_copy(s_hbm.at[pl.ds(s_id * nrt, nrt)],
                        stab.at[pl.ds(s_id * nrt, nrt)])
        pltpu.sync_copy(o_hbm.at[pl.ds(s_id * nrt, nrt)],
                        otab.at[pl.ds(s_id * nrt, nrt)])
        pltpu.sync_copy(pair_hbm.at[pl.ds(ebase, epw)], pvall)
        plsc.subcore_barrier()

        lanes = lax.iota(jnp.int32, 16)

        def issue(it, slot):
            base = it * ch
            for q in range(ch // 16):
                rows_ix = base + q * 16 + lanes
                ivc[slot, pl.ds(q * 16, 16)] = plsc.load_gather(
                    pvall, [rows_ix, jnp.zeros((16,), jnp.int32)])
                jvc[slot, pl.ds(q * 16, 16)] = plsc.load_gather(
                    pvall, [rows_ix, jnp.ones((16,), jnp.int32)])
            pltpu.async_copy(stab.at[ivc.at[slot]], sv.at[slot], gsem)
            pltpu.async_copy(otab.at[jvc.at[slot]], ov.at[slot], gsem)
            pltpu.async_copy(u_hbm.at[pl.ds(ebase + base, ch)],
                             uv.at[slot], usem)

        def drain(slot):
            pltpu.make_async_copy(stab.at[ivc.at[slot]],
                                  sv.at[slot], gsem).wait()
            pltpu.make_async_copy(otab.at[jvc.at[slot]],
                                  ov.at[slot], gsem).wait()
            pltpu.make_async_copy(u_hbm.at[pl.ds(0, ch)],
                                  uv.at[slot], usem).wait()

        issue(0, 0)

        def pair(g, carry):
            for b in range(2):
                it = g * 2 + b

                @pl.when(it + 1 < chn)
                def _():
                    issue(it + 1, 1 - b)

                drain(b)

                def row(r, c2):
                    for q in range(d // 16):
                        sl = pl.ds(q * 16, 16)
                        sv[b, r, sl] = sv[b, r, sl] * ov[b, r, sl] * uv[b, r, sl]
                    return c2

                lax.fori_loop(0, ch, row, 0)
                pltpu.sync_copy(sv.at[b], t_hbm.at[pl.ds(ebase + it * ch, ch)])
            return carry

        lax.fori_loop(0, chn // 2, pair, 0)

    mesh = plsc.VectorSubcoreMesh(core_axis_name="c", subcore_axis_name="s")
    fn = pl.kernel(
        body,
        out_type=jax.ShapeDtypeStruct((e, d), jnp.float32),
        mesh=mesh,
        compiler_params=pltpu.CompilerParams(use_tc_tiling_on_sc=False, needs_layout_passes=False),
        scratch_types=[
            pltpu.VMEM((epw, 2), jnp.int32),
            pltpu.VMEM((2, ch), jnp.int32),
            pltpu.VMEM((2, ch), jnp.int32),
            pltpu.VMEM((2, ch, d), jnp.float32),
            pltpu.VMEM((2, ch, d), jnp.float32),
            pltpu.VMEM((2, ch, d), jnp.float32),
            pltpu.VMEM_SHARED((n, d), jnp.float32),
            pltpu.VMEM_SHARED((n, d), jnp.float32),
            pltpu.SemaphoreType.DMA,
            pltpu.SemaphoreType.DMA,
        ],
    )
    return fn(s, o, u, pair)


def _attf_body(t_ref, ww_ref, bw_ref, out_ref):
    out_ref[...] = jnp.dot(t_ref[...], ww_ref[...],
                           precision=jax.lax.Precision.HIGHEST,
                           preferred_element_type=jnp.float32) + bw_ref[...]


def _atten_feats(t, Ww, bw):
    e, d = t.shape
    p = Ww.shape[1]
    blk = 2048
    return pl.pallas_call(
        _attf_body,
        grid=(e // blk,),
        in_specs=[
            pl.BlockSpec((blk, d), lambda i: (i, 0)),
            pl.BlockSpec((d, p), lambda i: (0, 0)),
            pl.BlockSpec((1, p), lambda i: (0, 0)),
        ],
        out_specs=pl.BlockSpec((blk, p), lambda i: (i, 0)),
        out_shape=jax.ShapeDtypeStruct((e, p), jnp.float32),
    )(t, Ww, bw.reshape(1, p))


def _scatter_dense(attf, pair, n, half, halves):
    e, p = attf.shape
    nh = n // halves
    rows = 64
    rsh = rows.bit_length() - 1
    nblk_per_sc = nh // rows // _NC
    sprows = rows * n
    ept = e // _NS
    zrows = 512

    def body(attf_hbm, pair_hbm, z_hbm, av, pv, xv, zerov, dix, dvals, spbuf):
        c = lax.axis_index("c")
        s = lax.axis_index("s")
        tb = s * ept
        pltpu.sync_copy(attf_hbm.at[pl.ds(tb, ept)], av)
        pltpu.sync_copy(pair_hbm.at[pl.ds(tb, ept)], pv)
        lanes = lax.iota(jnp.int32, 16)
        zl = jnp.zeros((16,), jnp.int32)
        ol = jnp.ones((16,), jnp.int32)

        def zrow(r, carry):
            zerov[r, :] = jnp.zeros((16,), jnp.float32)
            return carry

        lax.fori_loop(0, zrows, zrow, 0)

        def drow(r, carry):
            dvals[r, :] = jnp.full((16,), -1e4, jnp.float32)
            return carry

        lax.fori_loop(0, 16, drow, 0)

        for bb in range(nblk_per_sc):
            bl = c * nblk_per_sc + bb
            b = half * (nh // rows) + bl
            for q in range(sprows // _NS // zrows):
                pltpu.sync_copy(
                    zerov, spbuf.at[pl.ds(s * (sprows // _NS) + q * zrows,
                                          zrows)])
            plsc.subcore_barrier()

            dbase = sprows + (s << 8)

            def ixc(k, carry):
                for u2 in range(4):
                    sl = pl.ds((k * 4 + u2) * 16, 16)
                    rows_ix = (k * 4 + u2) * 16 + lanes
                    iv16 = plsc.load_gather(pv, [rows_ix, zl])
                    jv16 = plsc.load_gather(pv, [rows_ix, ol])
                    valid = (iv16 >> rsh) == b
                    loc = ((iv16 & (rows - 1)) << 10) | jv16
                    dummy = dbase + (((k * 4 + u2) * 16) & 255) + lanes
                    xv[sl] = jnp.where(valid, loc, dummy)
                return carry

            lax.fori_loop(0, ept // 64, ixc, 0)
            pltpu.sync_copy(av, spbuf.at[xv], add=True)
            i_loc = s * (rows // _NS) + lanes
            dloc = (i_loc << 10) + b * rows + i_loc
            dix[:] = jnp.where(lanes < rows // _NS, dloc, dbase + lanes)
            pltpu.sync_copy(dvals, spbuf.at[dix], add=True)
            plsc.subcore_barrier()
            pltpu.sync_copy(
                spbuf.at[pl.ds(s * (sprows // _NS), sprows // _NS)],
                z_hbm.at[pl.ds(bl * sprows + s * (sprows // _NS),
                               sprows // _NS)])

    mesh = plsc.VectorSubcoreMesh(core_axis_name="c", subcore_axis_name="s")
    fn = pl.kernel(
        body,
        out_type=jax.ShapeDtypeStruct((nh * n, p), jnp.float32),
        mesh=mesh,
        compiler_params=pltpu.CompilerParams(use_tc_tiling_on_sc=False,
                                             needs_layout_passes=False),
        scratch_types=[
            pltpu.VMEM((ept, p), jnp.float32),
            pltpu.VMEM((ept, 2), jnp.int32),
            pltpu.VMEM((ept,), jnp.int32),
            pltpu.VMEM((zrows, p), jnp.float32),
            pltpu.VMEM((16,), jnp.int32),
            pltpu.VMEM((16, p), jnp.float32),
            pltpu.VMEM_SHARED((sprows + _NS * 256, p), jnp.float32),
        ],
    )
    return fn(attf, pair)


def _softmax_rows(z2, n, p):
    npcols = z2.shape[1]
    bi = 16
    nch = npcols // 128

    def body(z_ref, o_ref):
        m = z_ref[:, 0:128]
        for k in range(1, nch):
            m = jnp.maximum(m, z_ref[:, k * 128:(k + 1) * 128])
        for sft in (16, 32, 64):
            m = jnp.maximum(
                m, jnp.concatenate([m[:, sft:], m[:, :sft]], axis=1))
        ssum = jnp.zeros((bi, 128), jnp.float32)
        for k in range(nch):
            ex = jnp.exp(z_ref[:, k * 128:(k + 1) * 128] - m)
            ssum = ssum + ex
            o_ref[:, k * 128:(k + 1) * 128] = ex
        for sft in (16, 32, 64):
            ssum = ssum + jnp.concatenate(
                [ssum[:, sft:], ssum[:, :sft]], axis=1)
        r = 1.0 / ssum
        for k in range(nch):
            o_ref[:, k * 128:(k + 1) * 128] = (
                o_ref[:, k * 128:(k + 1) * 128] * r)

    nr = z2.shape[0]
    return pl.pallas_call(
        body,
        grid=(nr // bi,),
        in_specs=[pl.BlockSpec((bi, npcols), lambda i: (i, 0))],
        out_specs=pl.BlockSpec((bi, npcols), lambda i: (i, 0)),
        out_shape=jax.ShapeDtypeStruct((nr, npcols), jnp.float32),
    )(z2)


def kernel(obj_feats, union_feats, pair_idxs, Ws, bs, Wo, bo, Ww, bw):
    n, d = obj_feats.shape
    p = Ww.shape[1]
    s, o = _project(obj_feats, Ws, bs, Wo, bo)
    t = _edge_product(s, o, union_feats, pair_idxs)
    attf = _atten_feats(t, Ww, bw)
    z = _scatter_dense(attf, pair_idxs, n, 0, 1)
    out = _softmax_rows(z.reshape(n, n * p), n, p)
    return out.reshape(n, n, p)
